# baseline (speedup 1.0000x reference)
import jax
import jax.numpy as jnp
from jax import lax
from jax.experimental import pallas as pl
from jax.experimental.pallas import tpu as pltpu
from jax.experimental.pallas import tpu_sc as plsc

NUM_EMB = 1_000_000
DIM = 32

NC = 2
NS = 16
NW = NC * NS

B = 16384
T = 50
BPW = B // NW
NBLK = BPW // 128


def _body(idx_hbm, table_hbm, out_hbm, idx_v, g_v, tb_v, gsem, osem):
    wid = lax.axis_index("s") * NC + lax.axis_index("c")
    b0 = wid * BPW

    pltpu.sync_copy(idx_hbm.at[:, pl.ds(b0, BPW)], idx_v)

    iota16 = lax.iota(jnp.int32, 16)
    iota32 = iota16 * DIM

    def unit(t, carry):
        copies = []
        for jb in range(NBLK):
            copies.append(
                pltpu.async_copy(
                    table_hbm.at[idx_v.at[t, pl.ds(jb * 128, 128)]],
                    g_v.at[jb],
                    gsem,
                )
            )
        for jb in range(NBLK):
            copies[jb].wait()
            for c in range(DIM):
                tr, r = c // 8, c % 8
                cols = iota16 * 0 + c
                for bl0 in range(0, 128, 16):
                    rows = iota16 + bl0
                    tb_v[tr, r, pl.ds(bl0, 16)] = plsc.load_gather(
                        g_v.at[jb], [rows, cols]
                    )
            ocopies = []
            for tr in range(4):
                ocopies.append(
                    pltpu.async_copy(
                        tb_v.at[tr],
                        out_hbm.at[t, tr, wid * NBLK + jb],
                        osem,
                    )
                )
            for oc in ocopies:
                oc.wait()
        return carry

    lax.fori_loop(0, T, unit, 0, unroll=False)


def kernel(token_ids, weight):
    w_lin = lax.optimization_barrier(weight.reshape(-1))
    w_row = w_lin.reshape(NUM_EMB, DIM)
    idx_t = token_ids.T

    mesh = plsc.VectorSubcoreMesh(core_axis_name="c", subcore_axis_name="s")
    fn = pl.kernel(
        _body,
        mesh=mesh,
        out_type=jax.ShapeDtypeStruct((T, 4, B // 128, 8, 128), jnp.float32),
        scratch_types=[
            pltpu.VMEM((T, BPW), jnp.int32),
            pltpu.VMEM((NBLK, 128, DIM), jnp.float32),
            pltpu.VMEM((4, 8, 128), jnp.float32),
            pltpu.SemaphoreType.DMA,
            pltpu.SemaphoreType.DMA,
        ],
        compiler_params=pltpu.CompilerParams(
            use_tc_tiling_on_sc=False, needs_layout_passes=False
        ),
    )
    out5 = fn(idx_t, w_row)
    return out5.transpose((2, 4, 0, 1, 3)).reshape(B, T, DIM)


# speedup vs baseline: 1.0002x; 1.0002x over previous
import jax
import jax.numpy as jnp
from jax import lax
from jax.experimental import pallas as pl
from jax.experimental.pallas import tpu as pltpu
from jax.experimental.pallas import tpu_sc as plsc

NUM_EMB = 1_000_000
DIM = 32

NC = 2
NS = 16
NW = NC * NS

B = 16384
T = 50
BPW = B // NW
NBLK = BPW // 128


def _body(idx_hbm, table_hbm, out_hbm, idx_v, g_v, tb_v, gsem, osem):
    wid = lax.axis_index("s") * NC + lax.axis_index("c")
    b0 = wid * BPW

    pltpu.sync_copy(idx_hbm.at[:, pl.ds(b0, BPW)], idx_v)

    iota16 = lax.iota(jnp.int32, 16)
    iota32 = iota16 * DIM

    def unit(t, carry):
        copies = []
        for jb in range(NBLK):
            copies.append(
                pltpu.async_copy(
                    table_hbm.at[idx_v.at[t, pl.ds(jb * 128, 128)]],
                    g_v.at[jb],
                    gsem,
                )
            )
        for jb in range(NBLK):
            copies[jb].wait()
            for c in range(DIM):
                tr, r = c // 8, c % 8
                cols = iota16 * 0 + c
                for bl0 in range(0, 128, 16):
                    rows = iota16 + bl0
                    tb_v[tr, r, pl.ds(bl0, 16)] = plsc.load_gather(
                        g_v.at[jb], [rows, cols]
                    )
            ocopies = []
            for tr in range(4):
                ocopies.append(
                    pltpu.async_copy(
                        tb_v.at[tr],
                        out_hbm.at[t, tr, wid * NBLK + jb],
                        osem,
                    )
                )
            for oc in ocopies:
                oc.wait()
        return carry

    lax.fori_loop(0, T, unit, 0, unroll=False)


def kernel(token_ids, weight):
    w_lin = lax.optimization_barrier(weight.reshape(-1))
    w_row = w_lin.reshape(NUM_EMB, DIM)
    idx_t = token_ids.T

    mesh = plsc.VectorSubcoreMesh(core_axis_name="c", subcore_axis_name="s")
    fn = pl.kernel(
        _body,
        mesh=mesh,
        out_type=jax.ShapeDtypeStruct((T, 4, B // 128, 8, 128), jnp.float32),
        scratch_types=[
            pltpu.VMEM((T, BPW), jnp.int32),
            pltpu.VMEM((NBLK, 128, DIM), jnp.float32),
            pltpu.VMEM((4, 8, 128), jnp.float32),
            pltpu.SemaphoreType.DMA,
            pltpu.SemaphoreType.DMA,
        ],
        compiler_params=pltpu.CompilerParams(
            use_tc_tiling_on_sc=False,
            needs_layout_passes=False,
            disable_bounds_checks=True,
        ),
    )
    out5 = fn(idx_t, w_row)
    return out5.transpose((2, 4, 0, 1, 3)).reshape(B, T, DIM)


# speedup vs baseline: 1.2864x; 1.2861x over previous
import jax
import jax.numpy as jnp
from jax import lax
from jax.experimental import pallas as pl
from jax.experimental.pallas import tpu as pltpu
from jax.experimental.pallas import tpu_sc as plsc

NUM_EMB = 1_000_000
DIM = 32

NC = 2
NS = 16
NW = NC * NS

B = 16384
T = 50
BPW = B // NW
NBLK = BPW // 128


def _body(idx_hbm, table_hbm, out_hbm, idx_v, g_v, tb_v, gsem, osem):
    wid = lax.axis_index("s") * NC + lax.axis_index("c")
    b0 = wid * BPW

    pltpu.sync_copy(idx_hbm.at[:, pl.ds(b0, BPW)], idx_v)

    iota16 = lax.iota(jnp.int32, 16)

    def unit(t, carry):
        copies = []
        for jb in range(NBLK):
            copies.append(
                pltpu.async_copy(
                    table_hbm.at[idx_v.at[t, pl.ds(jb * 128, 128)]],
                    g_v.at[jb],
                    gsem,
                )
            )
        ocopies = []
        for jb in range(NBLK):
            copies[jb].wait()

            @plsc.parallel_loop(0, DIM * 8, unroll=8)
            def _transpose(i):
                c = i >> 3
                blk = i & 7
                rows = iota16 + (blk << 4)
                cols = iota16 * 0 + c
                off = ((c >> 3) << 10) + ((c & 7) << 7) + (blk << 4)
                tb_v[jb, pl.ds(off, 16)] = plsc.load_gather(
                    g_v.at[jb], [rows, cols]
                )

            bblk = wid * NBLK + jb
            for tr in range(4):
                ocopies.append(
                    pltpu.async_copy(
                        tb_v.at[jb, pl.ds(tr * 1024, 1024)],
                        out_hbm.at[t, tr, bblk],
                        osem,
                    )
                )
        for oc in ocopies:
            oc.wait()
        return carry

    lax.fori_loop(0, T, unit, 0, unroll=False)


def kernel(token_ids, weight):
    w_lin = lax.optimization_barrier(weight.reshape(-1))
    w_row = w_lin.reshape(NUM_EMB, DIM)
    idx_t = token_ids.T

    mesh = plsc.VectorSubcoreMesh(core_axis_name="c", subcore_axis_name="s")
    fn = pl.kernel(
        _body,
        mesh=mesh,
        out_type=jax.ShapeDtypeStruct((T, 4, B // 128, 1024), jnp.float32),
        scratch_types=[
            pltpu.VMEM((T, BPW), jnp.int32),
            pltpu.VMEM((NBLK, 128, DIM), jnp.float32),
            pltpu.VMEM((NBLK, 4096), jnp.float32),
            pltpu.SemaphoreType.DMA,
            pltpu.SemaphoreType.DMA,
        ],
        compiler_params=pltpu.CompilerParams(
            use_tc_tiling_on_sc=False,
            needs_layout_passes=False,
            disable_bounds_checks=True,
        ),
    )
    out4 = fn(idx_t, w_row)
    out5 = out4.reshape(T, 4, B // 128, 8, 128)
    return out5.transpose((2, 4, 0, 1, 3)).reshape(B, T, DIM)


# speedup vs baseline: 2.0774x; 1.6149x over previous
import jax
import jax.numpy as jnp
from jax import lax
from jax.experimental import pallas as pl
from jax.experimental.pallas import tpu as pltpu
from jax.experimental.pallas import tpu_sc as plsc

NUM_EMB = 1_000_000
DIM = 32

NC = 2
NS = 16
NW = NC * NS

B = 16384
T = 50
BPW = B // NW
NBLK = BPW // 128


def _body(idx_hbm, table_hbm, out_hbm, idx_v, g_v, gp_v, tb_v, gsem, rsem, osem):
    wid = lax.axis_index("s") * NC + lax.axis_index("c")
    b0 = wid * BPW

    pltpu.sync_copy(idx_hbm.at[:, pl.ds(b0, BPW)], idx_v)

    iota16 = lax.iota(jnp.int32, 16)
    iota33 = iota16 * (DIM + 1)

    def unit(t, carry):
        copies = []
        for jb in range(NBLK):
            copies.append(
                pltpu.async_copy(
                    table_hbm.at[idx_v.at[t, pl.ds(jb * 128, 128)]],
                    g_v.at[jb],
                    gsem,
                )
            )
        ocopies = []
        for jb in range(NBLK):
            copies[jb].wait()
            @plsc.parallel_loop(0, 128, unroll=8)
            def _relayout(bl):
                src = g_v.at[jb]
                dst0 = bl * (DIM + 1)
                gp_v[jb, pl.ds(dst0, 16)] = src[bl, pl.ds(0, 16)]
                gp_v[jb, pl.ds(dst0 + 16, 16)] = src[bl, pl.ds(16, 16)]

            @plsc.parallel_loop(0, DIM * 8, unroll=8)
            def _transpose(i):
                c = i >> 3
                blk = i & 7
                idx = iota33 + ((blk << 4) * (DIM + 1) + c)
                off = ((c >> 3) << 10) + ((c & 7) << 7) + (blk << 4)
                tb_v[jb, pl.ds(off, 16)] = plsc.load_gather(
                    gp_v.at[jb], [idx]
                )

            bblk = wid * NBLK + jb
            for tr in range(4):
                ocopies.append(
                    pltpu.async_copy(
                        tb_v.at[jb, pl.ds(tr * 1024, 1024)],
                        out_hbm.at[t, tr, bblk],
                        osem,
                    )
                )
        for oc in ocopies:
            oc.wait()
        return carry

    lax.fori_loop(0, T, unit, 0, unroll=False)


def kernel(token_ids, weight):
    w_lin = lax.optimization_barrier(weight.reshape(-1))
    w_row = w_lin.reshape(NUM_EMB, DIM)
    idx_t = token_ids.T

    mesh = plsc.VectorSubcoreMesh(core_axis_name="c", subcore_axis_name="s")
    fn = pl.kernel(
        _body,
        mesh=mesh,
        out_type=jax.ShapeDtypeStruct((T, 4, B // 128, 1024), jnp.float32),
        scratch_types=[
            pltpu.VMEM((T, BPW), jnp.int32),
            pltpu.VMEM((NBLK, 128, DIM), jnp.float32),
            pltpu.VMEM((NBLK, 128 * (DIM + 1)), jnp.float32),
            pltpu.VMEM((NBLK, 4096), jnp.float32),
            pltpu.SemaphoreType.DMA,
            pltpu.SemaphoreType.DMA,
            pltpu.SemaphoreType.DMA,
        ],
        compiler_params=pltpu.CompilerParams(
            use_tc_tiling_on_sc=False,
            needs_layout_passes=False,
            disable_bounds_checks=True,
        ),
    )
    out4 = fn(idx_t, w_row)
    out5 = out4.reshape(T, 4, B // 128, 8, 128)
    return out5.transpose((2, 4, 0, 1, 3)).reshape(B, T, DIM)
